# baseline (speedup 1.0000x reference)
import functools

import jax
import jax.numpy as jnp
from jax import lax
from jax.experimental import pallas as pl
from jax.experimental.pallas import tpu as pltpu
from jax.experimental.pallas import tpu_sc as plsc

EPS = 1e-09
N_TASKS = 1000
N_SKILLS = 64
N_SPLITS = 8
BS = 16384
D = N_SKILLS * N_SPLITS

NUM_CORES = 2
NUM_SUBCORES = 16
NUM_WORKERS = NUM_CORES * NUM_SUBCORES
B_PER_W = BS // NUM_WORKERS
CHUNK = 128
N_CHUNKS = B_PER_W // CHUNK

T_PER_SUB = 64
LANES = 16

_mesh = plsc.VectorSubcoreMesh(core_axis_name="c", subcore_axis_name="s")


@functools.partial(
    pl.kernel,
    mesh=_mesh,
    out_type=[
        jax.ShapeDtypeStruct((BS, D), jnp.float32),
        jax.ShapeDtypeStruct((N_TASKS, D), jnp.float32),
    ],
    scratch_types=[
        pltpu.VMEM((T_PER_SUB, D), jnp.float32),
        pltpu.VMEM((B_PER_W,), jnp.int32),
        pltpu.VMEM((CHUNK, D), jnp.float32),
        pltpu.SemaphoreType.DMA,
    ],
    compiler_params=pltpu.CompilerParams(needs_layout_passes=False),
)
def _sc_run(idx_hbm, w_hbm, out_hbm, table_hbm, wv, idx_all, rows_v, sem):
    cid = lax.axis_index("c")
    sid = lax.axis_index("s")

    trow = pl.multiple_of(
        jnp.minimum(sid * T_PER_SUB, N_TASKS - T_PER_SUB), 8
    )
    pltpu.sync_copy(w_hbm.at[pl.ds(trow, T_PER_SUB)], wv)

    def norm_row(r, _):
        for g in range(N_SPLITS):
            base = g * N_SKILLS
            xs = [wv[r, pl.ds(base + j * LANES, LANES)] for j in range(4)]
            ss = [1.0 / (1.0 + jnp.exp(-x)) for x in xs]
            tot = jnp.sum(ss[0] + ss[1] + ss[2] + ss[3])
            inv = 1.0 / (jnp.full((LANES,), tot, jnp.float32) + EPS)
            for j in range(4):
                wv[r, pl.ds(base + j * LANES, LANES)] = ss[j] * inv
        return _

    lax.fori_loop(0, T_PER_SUB, norm_row, None)
    pltpu.sync_copy(wv, table_hbm.at[pl.ds(trow, T_PER_SUB)])
    plsc.subcore_barrier()

    wid = sid * NUM_CORES + cid
    bbase = wid * B_PER_W
    pltpu.sync_copy(idx_hbm.at[pl.ds(bbase, B_PER_W)], idx_all)
    for c in range(N_CHUNKS):
        off = c * CHUNK
        pltpu.async_copy(
            table_hbm.at[idx_all.at[pl.ds(off, CHUNK)]], rows_v, sem
        ).wait()
        pltpu.sync_copy(rows_v, out_hbm.at[pl.ds(bbase + off, CHUNK)])


B_BLK = 1024


def _repack_body(x_ref, o_ref):
    for s in range(N_SPLITS):
        o_ref[:, s, :] = x_ref[:, pl.ds(s * N_SKILLS, N_SKILLS)]


def _repack(out2d):
    return pl.pallas_call(
        _repack_body,
        grid=(BS // B_BLK,),
        in_specs=[pl.BlockSpec((B_BLK, D), lambda i: (i, 0))],
        out_specs=pl.BlockSpec((B_BLK, N_SPLITS, N_SKILLS), lambda i: (i, 0, 0)),
        out_shape=jax.ShapeDtypeStruct((BS, N_SPLITS, N_SKILLS), jnp.float32),
    )(out2d)


def kernel(routing_info, weights):
    idx = routing_info.reshape(BS).astype(jnp.int32)
    out, _ = _sc_run(idx, weights)
    return _repack(out)


# speedup vs baseline: 2.2419x; 2.2419x over previous
import functools

import jax
import jax.numpy as jnp
from jax import lax
from jax.experimental import pallas as pl
from jax.experimental.pallas import tpu as pltpu
from jax.experimental.pallas import tpu_sc as plsc

EPS = 1e-09
N_TASKS = 1000
N_SKILLS = 64
N_SPLITS = 8
BS = 16384
D = N_SKILLS * N_SPLITS

NUM_CORES = 2
NUM_SUBCORES = 16
NUM_WORKERS = NUM_CORES * NUM_SUBCORES
B_PER_W = BS // NUM_WORKERS
CHUNK = 128
N_CHUNKS = B_PER_W // CHUNK

T_PER_SUB = 64
LANES = 16

_mesh = plsc.VectorSubcoreMesh(core_axis_name="c", subcore_axis_name="s")


@functools.partial(
    pl.kernel,
    mesh=_mesh,
    out_type=[
        jax.ShapeDtypeStruct((BS, D), jnp.float32),
        jax.ShapeDtypeStruct((N_TASKS, D), jnp.float32),
    ],
    scratch_types=[
        pltpu.VMEM((T_PER_SUB, D), jnp.float32),
        pltpu.VMEM((B_PER_W,), jnp.int32),
        pltpu.VMEM((CHUNK, D), jnp.float32),
        pltpu.SemaphoreType.DMA,
    ],
    compiler_params=pltpu.CompilerParams(needs_layout_passes=False),
)
def _sc_run(idx_hbm, w_hbm, out_hbm, table_hbm, wv, idx_all, rows_v, sem):
    cid = lax.axis_index("c")
    sid = lax.axis_index("s")

    trow = pl.multiple_of(
        jnp.minimum(sid * T_PER_SUB, N_TASKS - T_PER_SUB), 8
    )
    pltpu.sync_copy(w_hbm.at[pl.ds(trow, T_PER_SUB)], wv)

    def norm_row(r, _):
        for g in range(N_SPLITS):
            base = g * N_SKILLS
            xs = [wv[r, pl.ds(base + j * LANES, LANES)] for j in range(4)]
            ss = [1.0 / (1.0 + jnp.exp(-x)) for x in xs]
            tot = jnp.sum(ss[0] + ss[1] + ss[2] + ss[3])
            inv = 1.0 / (jnp.full((LANES,), tot, jnp.float32) + EPS)
            for j in range(4):
                wv[r, pl.ds(base + j * LANES, LANES)] = ss[j] * inv
        return _

    lax.fori_loop(0, T_PER_SUB, norm_row, None)
    pltpu.sync_copy(wv, table_hbm.at[pl.ds(trow, T_PER_SUB)])
    plsc.subcore_barrier()

    wid = sid * NUM_CORES + cid
    bbase = wid * B_PER_W
    pltpu.sync_copy(idx_hbm.at[pl.ds(bbase, B_PER_W)], idx_all)
    for c in range(N_CHUNKS):
        off = c * CHUNK
        pltpu.async_copy(
            table_hbm.at[idx_all.at[pl.ds(off, CHUNK)]], rows_v, sem
        ).wait()
        pltpu.sync_copy(rows_v, out_hbm.at[pl.ds(bbase + off, CHUNK)])


def kernel(routing_info, weights):
    idx = routing_info.reshape(BS).astype(jnp.int32)
    out, _ = _sc_run(idx, weights)
    return out.reshape(BS, N_SPLITS, N_SKILLS)
